# baseline (speedup 1.0000x reference)
import functools

import jax
import jax.numpy as jnp
from jax.experimental import pallas as pl
from jax.experimental.pallas import tpu as pltpu

_BETA = 0.25


def _vq_body(nb, nk, hw, z_ref, cb_ref, zq_ref, idx_ref, loss_ref, sidx_ref):
    b = pl.program_id(0)
    cb = cb_ref[...]
    cnorm = jnp.sum(cb * cb, axis=1, keepdims=True)
    kiota = jax.lax.broadcasted_iota(jnp.int32, (nk, hw), 0)

    z = z_ref[0]
    s = jax.lax.dot_general(
        cb, z * -2.0, (((1,), (0,)), ((), ())),
        preferred_element_type=jnp.float32)
    znorm = jnp.sum(z * z, axis=0, keepdims=True)
    dist = (znorm + s) + cnorm

    m = jnp.min(dist, axis=0, keepdims=True)
    idx = jnp.min(jnp.where(dist == m, kiota, nk), axis=0, keepdims=True)
    sidx_ref[b] = idx

    onehot = (kiota == idx).astype(jnp.float32)
    zq = jax.lax.dot_general(
        cb, onehot, (((0,), (0,)), ((), ())),
        preferred_element_type=jnp.float32)

    d = zq - z
    zq_ref[0] = z + d
    part = jnp.sum(d * d, keepdims=True)

    @pl.when(b == 0)
    def _():
        loss_ref[...] = jnp.zeros_like(loss_ref)

    n = nb * z_ref.shape[1] * hw
    acc = loss_ref[...] + part

    @pl.when(b != nb - 1)
    def _():
        loss_ref[...] = acc

    @pl.when(b == nb - 1)
    def _():
        mean = acc / n
        loss_ref[...] = _BETA * mean + mean
        for bb in range(nb):
            idx_ref[:, bb * hw:(bb + 1) * hw] = sidx_ref[bb]


def kernel(z, codebook):
    B, C, H, W = z.shape
    K = codebook.shape[0]
    HW = H * W
    z3 = z.reshape(B, C, HW)

    zq3, idx2, loss11 = pl.pallas_call(
        functools.partial(_vq_body, B, K, HW),
        grid=(B,),
        in_specs=[
            pl.BlockSpec((1, C, HW), lambda b: (b, 0, 0)),
            pl.BlockSpec((K, C), lambda b: (0, 0)),
        ],
        out_specs=[
            pl.BlockSpec((1, C, HW), lambda b: (b, 0, 0)),
            pl.BlockSpec((1, B * HW), lambda b: (0, 0)),
            pl.BlockSpec((1, 1), lambda b: (0, 0)),
        ],
        out_shape=[
            jax.ShapeDtypeStruct((B, C, HW), jnp.float32),
            jax.ShapeDtypeStruct((1, B * HW), jnp.int32),
            jax.ShapeDtypeStruct((1, 1), jnp.float32),
        ],
        scratch_shapes=[pltpu.VMEM((B, 1, HW), jnp.int32)],
    )(z3, codebook)

    zq = zq3.reshape(B, C, H, W)
    idx = idx2.reshape(-1)
    loss = loss11.reshape(())
    return zq, idx, loss


# speedup vs baseline: 1.2463x; 1.2463x over previous
import functools

import jax
import jax.numpy as jnp
from jax.experimental import pallas as pl

_BETA = 0.25


def _vq_body(nt, nk, tc, z_ref, cb_ref, cn_ref, zq_ref, idx_ref, loss_ref):
    cb = cb_ref[...]
    cbb = cb.astype(jnp.bfloat16)
    cbm2 = cb * -2.0
    cnorm = cn_ref[...]

    kr = jax.lax.broadcasted_iota(jnp.int32, (1, nk), 1)
    arows = jnp.concatenate(
        [jnp.ones((1, nk), jnp.float32),
         (kr // 32).astype(jnp.float32),
         (kr % 32).astype(jnp.float32)], axis=0).astype(jnp.bfloat16)
    liota = jax.lax.broadcasted_iota(jnp.int32, (tc, nk), 1)

    acc = jnp.zeros((1, 1), jnp.float32)
    for c in range(nt // tc):
        t0 = c * tc
        z = z_ref[pl.ds(t0, tc), :]
        s = jax.lax.dot_general(
            z, cbm2, (((1,), (1,)), ((), ())),
            preferred_element_type=jnp.float32)
        znorm = jnp.sum(z * z, axis=1, keepdims=True)
        dist = (znorm + s) + cnorm

        m = jnp.min(dist, axis=1, keepdims=True)
        mask = (dist == m).astype(jnp.bfloat16)

        cnt3 = jax.lax.dot_general(
            arows, mask, (((1,), (1,)), ((), ())),
            preferred_element_type=jnp.float32)
        idxrow = (32.0 * cnt3[1:2] + cnt3[2:3]).astype(jnp.int32)
        idx_ref[:, pl.ds(t0, tc)] = idxrow

        zq = jax.lax.dot_general(
            mask, cbb, (((1,), (0,)), ((), ())),
            preferred_element_type=jnp.float32)
        zq_ref[pl.ds(t0, tc), :] = z + (zq - z)

        ties = jnp.max(cnt3[0:1]) > 1.5

        @pl.when(ties)
        def _():
            idxc = jnp.min(jnp.where(dist == m, liota, nk),
                           axis=1, keepdims=True)
            onehot = (liota == idxc).astype(jnp.bfloat16)
            c3 = jax.lax.dot_general(
                arows, onehot, (((1,), (1,)), ((), ())),
                preferred_element_type=jnp.float32)
            idx2 = (32.0 * c3[1:2] + c3[2:3]).astype(jnp.int32)
            idx_ref[:, pl.ds(t0, tc)] = idx2
            zq2 = jax.lax.dot_general(
                onehot, cbb, (((1,), (0,)), ((), ())),
                preferred_element_type=jnp.float32)
            zq_ref[pl.ds(t0, tc), :] = z + (zq2 - z)

        d = zq_ref[pl.ds(t0, tc), :] - z
        acc = acc + jnp.sum(d * d, keepdims=True)

    mean = acc / (nt * z_ref.shape[1])
    loss_ref[...] = _BETA * mean + mean


def kernel(z, codebook):
    B, C, H, W = z.shape
    K = codebook.shape[0]
    NT = B * H * W
    TC = 512
    zf = jnp.transpose(z, (0, 2, 3, 1)).reshape(NT, C)
    cn = jnp.sum(codebook ** 2, axis=1)[None, :]

    zqf, idx2, loss11 = pl.pallas_call(
        functools.partial(_vq_body, NT, K, TC),
        grid=(1,),
        in_specs=[
            pl.BlockSpec((NT, C), lambda i: (0, 0)),
            pl.BlockSpec((K, C), lambda i: (0, 0)),
            pl.BlockSpec((1, K), lambda i: (0, 0)),
        ],
        out_specs=[
            pl.BlockSpec((NT, C), lambda i: (0, 0)),
            pl.BlockSpec((1, NT), lambda i: (0, 0)),
            pl.BlockSpec((1, 1), lambda i: (0, 0)),
        ],
        out_shape=[
            jax.ShapeDtypeStruct((NT, C), jnp.float32),
            jax.ShapeDtypeStruct((1, NT), jnp.int32),
            jax.ShapeDtypeStruct((1, 1), jnp.float32),
        ],
    )(zf, codebook, cn)

    zq = jnp.transpose(zqf.reshape(B, H, W, C), (0, 3, 1, 2))
    idx = idx2.reshape(-1)
    loss = loss11.reshape(())
    return zq, idx, loss


# speedup vs baseline: 1.4060x; 1.1281x over previous
import functools

import jax
import jax.numpy as jnp
from jax.experimental import pallas as pl

_BETA = 0.25


def _vq_body(nt, nk, tc, z_ref, cb_ref, cn_ref, zq_ref, idx_ref, loss_ref):
    cb = cb_ref[...]
    cbb = cb.astype(jnp.bfloat16)
    cbm2 = cb * -2.0
    cnorm = cn_ref[...]

    kr = jax.lax.broadcasted_iota(jnp.int32, (1, nk), 1)
    arows = jnp.concatenate(
        [(kr // 32).astype(jnp.float32),
         (kr % 32).astype(jnp.float32)], axis=0).astype(jnp.bfloat16)
    liota = jax.lax.broadcasted_iota(jnp.int32, (tc, nk), 1)

    acc = jnp.zeros((1, 1), jnp.float32)
    for c in range(nt // tc):
        t0 = c * tc
        z = z_ref[pl.ds(t0, tc), :]
        s = jax.lax.dot_general(
            z, cbm2, (((1,), (1,)), ((), ())),
            preferred_element_type=jnp.float32)
        znorm = jnp.sum(z * z, axis=1, keepdims=True)
        dist = (znorm + s) + cnorm

        m = jnp.min(dist, axis=1, keepdims=True)
        idxc = jnp.min(jnp.where(dist == m, liota, nk),
                       axis=1, keepdims=True)
        onehot = (liota == idxc).astype(jnp.bfloat16)

        hilo = jax.lax.dot_general(
            arows, onehot, (((1,), (1,)), ((), ())),
            preferred_element_type=jnp.float32)
        idxrow = (32.0 * hilo[0:1] + hilo[1:2]).astype(jnp.int32)
        idx_ref[:, pl.ds(t0, tc)] = idxrow

        zq = jax.lax.dot_general(
            onehot, cbb, (((1,), (0,)), ((), ())),
            preferred_element_type=jnp.float32)
        d = zq - z
        zq_ref[pl.ds(t0, tc), :] = z + d
        acc = acc + jnp.sum(d * d, keepdims=True)

    mean = acc / (nt * z_ref.shape[1])
    loss_ref[...] = _BETA * mean + mean


def kernel(z, codebook):
    B, C, H, W = z.shape
    K = codebook.shape[0]
    NT = B * H * W
    TC = 512
    zf = jnp.transpose(z, (0, 2, 3, 1)).reshape(NT, C)
    cn = jnp.sum(codebook ** 2, axis=1)[None, :]

    zqf, idx2, loss11 = pl.pallas_call(
        functools.partial(_vq_body, NT, K, TC),
        grid=(1,),
        in_specs=[
            pl.BlockSpec((NT, C), lambda i: (0, 0)),
            pl.BlockSpec((K, C), lambda i: (0, 0)),
            pl.BlockSpec((1, K), lambda i: (0, 0)),
        ],
        out_specs=[
            pl.BlockSpec((NT, C), lambda i: (0, 0)),
            pl.BlockSpec((1, NT), lambda i: (0, 0)),
            pl.BlockSpec((1, 1), lambda i: (0, 0)),
        ],
        out_shape=[
            jax.ShapeDtypeStruct((NT, C), jnp.float32),
            jax.ShapeDtypeStruct((1, NT), jnp.int32),
            jax.ShapeDtypeStruct((1, 1), jnp.float32),
        ],
    )(zf, codebook, cn)

    zq = jnp.transpose(zqf.reshape(B, H, W, C), (0, 3, 1, 2))
    idx = idx2.reshape(-1)
    loss = loss11.reshape(())
    return zq, idx, loss
